# baseline (speedup 1.0000x reference)
import jax
import jax.numpy as jnp
import numpy as np
from jax.experimental import pallas as pl
from jax.experimental.pallas import tpu as pltpu

_DELTA = 1e-08
_P = 3
_Q = 3
_NCX = 35
_NCY = 35
_DIM = 3
_U = 512
_V = 512


def _knots(n_ctrl, degree):
    interior = n_ctrl - degree - 1
    return np.concatenate([
        np.zeros(degree + 1),
        np.arange(1, interior + 1) / float(interior + 1),
        np.ones(degree + 1),
    ]).astype(np.float64)


def _spans(n_ctrl, degree, pts, kv):
    spans = np.searchsorted(kv, pts, side='right') - 1
    spans = np.where(pts == kv[n_ctrl], n_ctrl - 1, spans)
    return spans.astype(np.int64)


def _basis(span, pts, degree, kv):
    n = len(pts)
    Nb = np.empty((n, degree + 1), dtype=np.float64)
    Nb[:, 0] = 1.0
    left = np.empty((n, degree + 1), dtype=np.float64)
    right = np.empty((n, degree + 1), dtype=np.float64)
    for j in range(1, degree + 1):
        left[:, j] = pts - kv[span + 1 - j]
        right[:, j] = kv[span + j] - pts
        saved = np.zeros(n, dtype=np.float64)
        for r in range(0, j):
            temp = Nb[:, r] / (right[:, r + 1] + left[:, j - r])
            Nb[:, r] = saved + right[:, r + 1] * temp
            saved = left[:, j - r] * temp
        Nb[:, j] = saved
    return Nb


def _dense_basis_matrix(n_ctrl, degree, npts):
    kv = _knots(n_ctrl, degree)
    pts = np.linspace(0.0 + _DELTA, 1.0 - _DELTA, npts)
    span = _spans(n_ctrl, degree, pts, kv)
    Nb = _basis(span, pts, degree, kv)
    idx = span[:, None] - degree + np.arange(degree + 1)[None, :]
    A = np.zeros((npts, n_ctrl), dtype=np.float64)
    np.put_along_axis(A, idx, Nb, axis=1)
    return A.astype(np.float32)


_AU = _dense_basis_matrix(_NCX, _P, _U)
_AVT = _dense_basis_matrix(_NCY, _Q, _V).T.copy()


_BT = 1


def _surf_kernel(ctrl_ref, au_ref, avt_ref, out_ref):
    au = au_ref[...]
    avt = avt_ref[...]
    for b in range(_BT):
        tw = jnp.dot(au, ctrl_ref[b, _DIM], preferred_element_type=jnp.float32)
        w_r = 1.0 / jnp.dot(tw, avt, preferred_element_type=jnp.float32)
        for d in range(_DIM):
            c = ctrl_ref[b, d]
            t = jnp.dot(au, c, preferred_element_type=jnp.float32)
            out_ref[b, d] = jnp.dot(t, avt, preferred_element_type=jnp.float32) * w_r


@jax.jit
def kernel(input):
    B = input.shape[0]
    ctrl = jnp.transpose(input, (0, 3, 1, 2))
    out = pl.pallas_call(
        _surf_kernel,
        grid=(B // _BT,),
        in_specs=[
            pl.BlockSpec((_BT, _DIM + 1, _NCX, _NCY), lambda b: (b, 0, 0, 0)),
            pl.BlockSpec((_U, _NCX), lambda b: (0, 0)),
            pl.BlockSpec((_NCY, _V), lambda b: (0, 0)),
        ],
        out_specs=pl.BlockSpec((_BT, _DIM, _U, _V), lambda b: (b, 0, 0, 0)),
        out_shape=jax.ShapeDtypeStruct((B, _DIM, _U, _V), jnp.float32),
        compiler_params=pltpu.CompilerParams(
            dimension_semantics=("arbitrary",),
        ),
    )(ctrl, jnp.asarray(_AU), jnp.asarray(_AVT))
    return jnp.transpose(out, (0, 2, 3, 1))


# speedup vs baseline: 1.0350x; 1.0350x over previous
import jax
import jax.numpy as jnp
import numpy as np
from jax.experimental import pallas as pl
from jax.experimental.pallas import tpu as pltpu

_DELTA = 1e-08
_P = 3
_Q = 3
_NCX = 35
_NCY = 35
_DIM = 3
_U = 512
_V = 512


def _knots(n_ctrl, degree):
    interior = n_ctrl - degree - 1
    return np.concatenate([
        np.zeros(degree + 1),
        np.arange(1, interior + 1) / float(interior + 1),
        np.ones(degree + 1),
    ]).astype(np.float64)


def _spans(n_ctrl, degree, pts, kv):
    spans = np.searchsorted(kv, pts, side='right') - 1
    spans = np.where(pts == kv[n_ctrl], n_ctrl - 1, spans)
    return spans.astype(np.int64)


def _basis(span, pts, degree, kv):
    n = len(pts)
    Nb = np.empty((n, degree + 1), dtype=np.float64)
    Nb[:, 0] = 1.0
    left = np.empty((n, degree + 1), dtype=np.float64)
    right = np.empty((n, degree + 1), dtype=np.float64)
    for j in range(1, degree + 1):
        left[:, j] = pts - kv[span + 1 - j]
        right[:, j] = kv[span + j] - pts
        saved = np.zeros(n, dtype=np.float64)
        for r in range(0, j):
            temp = Nb[:, r] / (right[:, r + 1] + left[:, j - r])
            Nb[:, r] = saved + right[:, r + 1] * temp
            saved = left[:, j - r] * temp
        Nb[:, j] = saved
    return Nb


def _dense_basis_matrix(n_ctrl, degree, npts):
    kv = _knots(n_ctrl, degree)
    pts = np.linspace(0.0 + _DELTA, 1.0 - _DELTA, npts)
    span = _spans(n_ctrl, degree, pts, kv)
    Nb = _basis(span, pts, degree, kv)
    idx = span[:, None] - degree + np.arange(degree + 1)[None, :]
    A = np.zeros((npts, n_ctrl), dtype=np.float64)
    np.put_along_axis(A, idx, Nb, axis=1)
    return A.astype(np.float32)


_AU = _dense_basis_matrix(_NCX, _P, _U)
_AVT = _dense_basis_matrix(_NCY, _Q, _V).T.copy()


_BT = 2


def _surf_kernel(ctrl_ref, au_ref, avt_ref, out_ref):
    au = au_ref[...]
    avt = avt_ref[...]
    for b in range(_BT):
        tw = jnp.dot(au, ctrl_ref[b, _DIM], preferred_element_type=jnp.float32)
        w_r = 1.0 / jnp.dot(tw, avt, preferred_element_type=jnp.float32)
        for d in range(_DIM):
            c = ctrl_ref[b, d]
            t = jnp.dot(au, c, preferred_element_type=jnp.float32)
            out_ref[b, d] = jnp.dot(t, avt, preferred_element_type=jnp.float32) * w_r


@jax.jit
def kernel(input):
    B = input.shape[0]
    ctrl = jnp.transpose(input, (0, 3, 1, 2))
    out = pl.pallas_call(
        _surf_kernel,
        grid=(B // _BT,),
        in_specs=[
            pl.BlockSpec((_BT, _DIM + 1, _NCX, _NCY), lambda b: (b, 0, 0, 0)),
            pl.BlockSpec((_U, _NCX), lambda b: (0, 0)),
            pl.BlockSpec((_NCY, _V), lambda b: (0, 0)),
        ],
        out_specs=pl.BlockSpec((_BT, _DIM, _U, _V), lambda b: (b, 0, 0, 0)),
        out_shape=jax.ShapeDtypeStruct((B, _DIM, _U, _V), jnp.float32),
        compiler_params=pltpu.CompilerParams(
            dimension_semantics=("arbitrary",),
        ),
    )(ctrl, jnp.asarray(_AU), jnp.asarray(_AVT))
    return jnp.transpose(out, (0, 2, 3, 1))
